# baseline (speedup 1.0000x reference)
import functools

import jax
import jax.numpy as jnp
from jax import lax
from jax.experimental import pallas as pl
from jax.experimental.pallas import tpu as pltpu
from jax.experimental.pallas import tpu_sc as plsc

NC = 2
NS = 16
L = 16
NW = NC * NS

BATCH, SEQ, D, K, V = 16384, 200, 64, 2, 10
N = BATCH * SEQ
NBH = BATCH // 128
BH_PER_W = NBH // NW
SLAB_IN = 128 * SEQ
SLAB_OUT = SEQ * 2 * 128


def _sc_body(x_hbm, emb_hbm, wb_hbm, out_hbm, emb_v, wb_v, tab_v,
             xbuf0, xbuf1, obuf, in_sem0, in_sem1, out_sem0, out_sem1):
    in_sems = (in_sem0, in_sem1)
    cid = lax.axis_index("c")
    sid = lax.axis_index("s")
    wid = sid * NC + cid
    iota = lax.iota(jnp.int32, L)

    pltpu.sync_copy(emb_hbm, emb_v)
    pltpu.sync_copy(wb_hbm, wb_v)

    vrow = iota * D
    mlt = iota < V
    t0 = jnp.zeros((L,), jnp.float32)
    t1 = jnp.zeros((L,), jnp.float32)
    for d in range(D):
        ev = plsc.load_gather(emb_v, [vrow + d], mask=mlt)
        w0 = plsc.load_gather(wb_v, [jnp.full((L,), 1 + 2 * d, jnp.int32)])
        w1 = plsc.load_gather(wb_v, [jnp.full((L,), 2 + 2 * d, jnp.int32)])
        t0 = t0 + ev * w0
        t1 = t1 + ev * w1
    t0 = t0 + plsc.load_gather(wb_v, [jnp.full((L,), 1 + 2 * D, jnp.int32)])
    t1 = t1 + plsc.load_gather(wb_v, [jnp.full((L,), 2 + 2 * D, jnp.int32)])
    plsc.store_scatter(tab_v, [iota * 2], t0, mask=mlt)
    plsc.store_scatter(tab_v, [iota * 2 + 1], t1, mask=mlt)

    rg = [16 * g + iota for g in range(8)]
    xbufs = (xbuf0, xbuf1)
    HALF = 96

    def half_body(xb, lo, hi):
        def body(l):
            lc = jnp.full((L,), 0, jnp.int32) + l
            for g in range(8):
                xv = plsc.load_gather(xb, [rg[g], lc])
                x2 = xv * 2
                ov0 = plsc.load_gather(tab_v, [x2])
                ov1 = plsc.load_gather(tab_v, [x2 + 1])
                obuf[l, pl.ds(16 * g, L)] = ov0
                obuf[l, pl.ds(128 + 16 * g, L)] = ov1
        plsc.parallel_loop(lo, hi, unroll=2)(body)

    def start_in(bi):
        bhi = wid * BH_PER_W + bi
        return pltpu.async_copy(
            x_hbm.at[pl.ds(bhi * 128, 128), :], xbufs[bi % 2],
            in_sems[bi % 2])

    out0 = out1 = None
    pend_in = start_in(0)
    for bi in range(BH_PER_W):
        bhi = wid * BH_PER_W + bi
        nxt = start_in(bi + 1) if bi + 1 < BH_PER_W else None
        pend_in.wait()
        pend_in = nxt
        if out0 is not None:
            out0.wait()
            out1.wait()
        xb = xbufs[bi % 2]
        half_body(xb, 0, HALF)
        out0 = pltpu.async_copy(
            obuf.at[pl.ds(0, HALF), :],
            out_hbm.at[pl.ds(0, HALF), pl.ds(bhi * 256, 256)], out_sem0)
        half_body(xb, HALF, SEQ)
        out1 = pltpu.async_copy(
            obuf.at[pl.ds(HALF, SEQ - HALF), :],
            out_hbm.at[pl.ds(HALF, SEQ - HALF), pl.ds(bhi * 256, 256)],
            out_sem1)
    out0.wait()
    out1.wait()


_sc_lookup = functools.partial(
    pl.kernel,
    out_type=jax.ShapeDtypeStruct((SEQ, BATCH * 2), jnp.float32),
    mesh=plsc.VectorSubcoreMesh(core_axis_name="c", subcore_axis_name="s"),
    compiler_params=pltpu.CompilerParams(
        needs_layout_passes=False, use_tc_tiling_on_sc=True),
    scratch_types=[
        pltpu.VMEM((V * D,), jnp.float32),
        pltpu.VMEM((1 + D * K + 15,), jnp.float32),
        pltpu.VMEM((2 * L,), jnp.float32),
        pltpu.VMEM((128, SEQ), jnp.int32),
        pltpu.VMEM((128, SEQ), jnp.int32),
        pltpu.VMEM((SEQ, 256), jnp.float32),
        pltpu.SemaphoreType.DMA,
        pltpu.SemaphoreType.DMA,
        pltpu.SemaphoreType.DMA,
        pltpu.SemaphoreType.DMA,
    ],
)(_sc_body)


def kernel(x, emb, W, b):
    xf = x.astype(jnp.int32)
    embf = emb.reshape(-1)
    wb = jnp.concatenate(
        [jnp.zeros((1,), jnp.float32), W.reshape(-1), b,
         jnp.zeros((13,), jnp.float32)])
    out2 = _sc_lookup(xf, embf, wb)
    out4 = out2.reshape(SEQ, NBH, 2, 128)
    return out4.transpose(1, 3, 0, 2).reshape(BATCH, SEQ, K)


# speedup vs baseline: 1.0953x; 1.0953x over previous
import functools

import jax
import jax.numpy as jnp
from jax import lax
from jax.experimental import pallas as pl
from jax.experimental.pallas import tpu as pltpu
from jax.experimental.pallas import tpu_sc as plsc

NC = 2
NS = 16
L = 16
NW = NC * NS

BATCH, SEQ, D, K, V = 16384, 200, 64, 2, 10
N = BATCH * SEQ
NBH = BATCH // 128
BH_PER_W = NBH // NW
SLAB_IN = 128 * SEQ
SLAB_OUT = SEQ * 2 * 128


def _sc_body(x_hbm, emb_hbm, wb_hbm, out_hbm, emb_v, wb_v, tab_v,
             xbuf0, xbuf1, obuf, in_sem0, in_sem1, out_sem0, out_sem1):
    in_sems = (in_sem0, in_sem1)
    cid = lax.axis_index("c")
    sid = lax.axis_index("s")
    wid = sid * NC + cid
    iota = lax.iota(jnp.int32, L)

    pltpu.sync_copy(emb_hbm, emb_v)
    pltpu.sync_copy(wb_hbm, wb_v)

    vrow = iota * D
    mlt = iota < V
    t0 = jnp.zeros((L,), jnp.float32)
    t1 = jnp.zeros((L,), jnp.float32)
    for d in range(D):
        ev = plsc.load_gather(emb_v, [vrow + d], mask=mlt)
        w0 = plsc.load_gather(wb_v, [jnp.full((L,), 1 + 2 * d, jnp.int32)])
        w1 = plsc.load_gather(wb_v, [jnp.full((L,), 2 + 2 * d, jnp.int32)])
        t0 = t0 + ev * w0
        t1 = t1 + ev * w1
    t0 = t0 + plsc.load_gather(wb_v, [jnp.full((L,), 1 + 2 * D, jnp.int32)])
    t1 = t1 + plsc.load_gather(wb_v, [jnp.full((L,), 2 + 2 * D, jnp.int32)])
    plsc.store_scatter(tab_v, [iota * 2], t0, mask=mlt)
    plsc.store_scatter(tab_v, [iota * 2 + 1], t1, mask=mlt)

    cg = [(16 * g + iota) * SEQ for g in range(8)]
    xbufs = (xbuf0, xbuf1)
    HALF = 96

    def half_body(xb, lo, hi):
        def body(l):
            for g in range(8):
                xv = plsc.load_gather(xb, [cg[g] + l])
                x2 = xv * 2
                ov0 = plsc.load_gather(tab_v, [x2])
                ov1 = plsc.load_gather(tab_v, [x2 + 1])
                obuf[l, pl.ds(16 * g, L)] = ov0
                obuf[l, pl.ds(128 + 16 * g, L)] = ov1
        plsc.parallel_loop(lo, hi, unroll=4)(body)

    def start_in(bi):
        bhi = wid * BH_PER_W + bi
        return pltpu.async_copy(
            x_hbm.at[pl.ds(bhi * SLAB_IN, SLAB_IN)], xbufs[bi % 2],
            in_sems[bi % 2])

    out0 = out1 = None
    pend_in = start_in(0)
    for bi in range(BH_PER_W):
        bhi = wid * BH_PER_W + bi
        nxt = start_in(bi + 1) if bi + 1 < BH_PER_W else None
        pend_in.wait()
        pend_in = nxt
        if out0 is not None:
            out0.wait()
            out1.wait()
        xb = xbufs[bi % 2]
        half_body(xb, 0, HALF)
        out0 = pltpu.async_copy(
            obuf.at[pl.ds(0, HALF), :],
            out_hbm.at[pl.ds(0, HALF), pl.ds(bhi * 256, 256)], out_sem0)
        half_body(xb, HALF, SEQ)
        out1 = pltpu.async_copy(
            obuf.at[pl.ds(HALF, SEQ - HALF), :],
            out_hbm.at[pl.ds(HALF, SEQ - HALF), pl.ds(bhi * 256, 256)],
            out_sem1)
    out0.wait()
    out1.wait()


_sc_lookup = functools.partial(
    pl.kernel,
    out_type=jax.ShapeDtypeStruct((SEQ, BATCH * 2), jnp.float32),
    mesh=plsc.VectorSubcoreMesh(core_axis_name="c", subcore_axis_name="s"),
    compiler_params=pltpu.CompilerParams(needs_layout_passes=False),
    scratch_types=[
        pltpu.VMEM((V * D,), jnp.float32),
        pltpu.VMEM((1 + D * K + 15,), jnp.float32),
        pltpu.VMEM((2 * L,), jnp.float32),
        pltpu.VMEM((SLAB_IN,), jnp.int32),
        pltpu.VMEM((SLAB_IN,), jnp.int32),
        pltpu.VMEM((SEQ, 256), jnp.float32),
        pltpu.SemaphoreType.DMA,
        pltpu.SemaphoreType.DMA,
        pltpu.SemaphoreType.DMA,
        pltpu.SemaphoreType.DMA,
    ],
)(_sc_body)


def kernel(x, emb, W, b):
    xf = x.reshape(-1).astype(jnp.int32)
    embf = emb.reshape(-1)
    wb = jnp.concatenate(
        [jnp.zeros((1,), jnp.float32), W.reshape(-1), b,
         jnp.zeros((13,), jnp.float32)])
    out2 = _sc_lookup(xf, embf, wb)
    out4 = out2.reshape(SEQ, NBH, 2, 128)
    return out4.transpose(1, 3, 0, 2).reshape(BATCH, SEQ, K)


# speedup vs baseline: 1.1248x; 1.0269x over previous
import functools

import jax
import jax.numpy as jnp
from jax import lax
from jax.experimental import pallas as pl
from jax.experimental.pallas import tpu as pltpu
from jax.experimental.pallas import tpu_sc as plsc

NC = 2
NS = 16
L = 16
NW = NC * NS

BATCH, SEQ, D, K, V = 16384, 200, 64, 2, 10
N = BATCH * SEQ
NBH = BATCH // 128
BH_PER_W = NBH // NW
SLAB_IN = 128 * SEQ
SLAB_OUT = SEQ * 2 * 128


def _sc_body(x_hbm, emb_hbm, wb_hbm, out_hbm, emb_v, wb_v, tab_v,
             xbuf0, xbuf1, obuf, in_sem0, in_sem1, out_sem0, out_sem1):
    in_sems = (in_sem0, in_sem1)
    cid = lax.axis_index("c")
    sid = lax.axis_index("s")
    wid = sid * NC + cid
    iota = lax.iota(jnp.int32, L)

    first_in = pltpu.async_copy(
        x_hbm.at[pl.ds(wid * BH_PER_W * SLAB_IN, SLAB_IN)], xbuf0, in_sem0)
    pltpu.sync_copy(emb_hbm, emb_v)
    pltpu.sync_copy(wb_hbm, wb_v)

    vrow = iota * D
    mlt = iota < V
    t0 = jnp.zeros((L,), jnp.float32)
    t1 = jnp.zeros((L,), jnp.float32)
    for d in range(D):
        ev = plsc.load_gather(emb_v, [vrow + d], mask=mlt)
        w0 = plsc.load_gather(wb_v, [jnp.full((L,), 1 + 2 * d, jnp.int32)])
        w1 = plsc.load_gather(wb_v, [jnp.full((L,), 2 + 2 * d, jnp.int32)])
        t0 = t0 + ev * w0
        t1 = t1 + ev * w1
    t0 = t0 + plsc.load_gather(wb_v, [jnp.full((L,), 1 + 2 * D, jnp.int32)])
    t1 = t1 + plsc.load_gather(wb_v, [jnp.full((L,), 2 + 2 * D, jnp.int32)])
    plsc.store_scatter(tab_v, [iota * 2], t0, mask=mlt)
    plsc.store_scatter(tab_v, [iota * 2 + 1], t1, mask=mlt)

    cg = [(16 * g + iota) * SEQ for g in range(8)]
    xbufs = (xbuf0, xbuf1)
    HALF = 96

    def half_body(xb, lo, hi):
        def body(l):
            for g in range(8):
                xv = plsc.load_gather(xb, [cg[g] + l])
                x2 = xv * 2
                ov0 = plsc.load_gather(tab_v, [x2])
                ov1 = plsc.load_gather(tab_v, [x2 + 1])
                obuf[l, pl.ds(16 * g, L)] = ov0
                obuf[l, pl.ds(128 + 16 * g, L)] = ov1
        plsc.parallel_loop(lo, hi, unroll=2)(body)

    def start_in(bi):
        bhi = wid * BH_PER_W + bi
        return pltpu.async_copy(
            x_hbm.at[pl.ds(bhi * SLAB_IN, SLAB_IN)], xbufs[bi % 2],
            in_sems[bi % 2])

    out0 = out1 = None
    pend_in = first_in
    for bi in range(BH_PER_W):
        bhi = wid * BH_PER_W + bi
        nxt = start_in(bi + 1) if bi + 1 < BH_PER_W else None
        pend_in.wait()
        pend_in = nxt
        if out0 is not None:
            out0.wait()
            out1.wait()
        xb = xbufs[bi % 2]
        half_body(xb, 0, HALF)
        out0 = pltpu.async_copy(
            obuf.at[pl.ds(0, HALF), :],
            out_hbm.at[pl.ds(0, HALF), pl.ds(bhi * 256, 256)], out_sem0)
        half_body(xb, HALF, SEQ)
        out1 = pltpu.async_copy(
            obuf.at[pl.ds(HALF, SEQ - HALF), :],
            out_hbm.at[pl.ds(HALF, SEQ - HALF), pl.ds(bhi * 256, 256)],
            out_sem1)
    out0.wait()
    out1.wait()


_sc_lookup = functools.partial(
    pl.kernel,
    out_type=jax.ShapeDtypeStruct((SEQ, BATCH * 2), jnp.float32),
    mesh=plsc.VectorSubcoreMesh(core_axis_name="c", subcore_axis_name="s"),
    compiler_params=pltpu.CompilerParams(needs_layout_passes=False),
    scratch_types=[
        pltpu.VMEM((V * D,), jnp.float32),
        pltpu.VMEM((1 + D * K + 15,), jnp.float32),
        pltpu.VMEM((2 * L,), jnp.float32),
        pltpu.VMEM((SLAB_IN,), jnp.int32),
        pltpu.VMEM((SLAB_IN,), jnp.int32),
        pltpu.VMEM((SEQ, 256), jnp.float32),
        pltpu.SemaphoreType.DMA,
        pltpu.SemaphoreType.DMA,
        pltpu.SemaphoreType.DMA,
        pltpu.SemaphoreType.DMA,
    ],
)(_sc_body)


def kernel(x, emb, W, b):
    xf = x.reshape(-1).astype(jnp.int32)
    embf = emb.reshape(-1)
    wb = jnp.concatenate(
        [jnp.zeros((1,), jnp.float32), W.reshape(-1), b,
         jnp.zeros((13,), jnp.float32)])
    out2 = _sc_lookup(xf, embf, wb)
    out4 = out2.reshape(SEQ, NBH, 2, 128)
    return out4.transpose(1, 3, 0, 2).reshape(BATCH, SEQ, K)
